# baseline (speedup 1.0000x reference)
import functools

import jax
import jax.numpy as jnp
from jax import lax
from jax.experimental import pallas as pl
from jax.experimental.pallas import tpu as pltpu
from jax.experimental.pallas import tpu_sc as plsc

_COEFF = 0.5
_THRESH = 0.3
_EPS = 1e-12
_BLK = 2000
_BIG = 2**30


def _topk_body(res_ref, ec_ref, rc_ref, ei_ref, ri_ref, sc_ref,
               m_ref, ab_ref, resn_ref):
    i = pl.program_id(0)
    nb = pl.num_programs(0)
    blk = ec_ref.shape[0]
    q = res_ref.shape[0]
    nr = rc_ref.shape[0]

    @pl.when(i == 0)
    def _init():
        res = res_ref[...]
        n = jnp.sqrt(jnp.sum(res * res, axis=1, keepdims=True))
        resn_ref[...] = res / jnp.maximum(n, _EPS)
        m_ref[...] = jnp.full(m_ref.shape, -jnp.inf, jnp.float32)
        ab_ref[...] = jnp.zeros(ab_ref.shape, jnp.int32)

    ec = ec_ref[...]
    n = jnp.sqrt(jnp.sum(ec * ec, axis=1, keepdims=True))
    ecn = ec / jnp.maximum(n, _EPS)
    resn = resn_ref[...]
    sim = lax.dot_general(ecn, resn, (((1,), (1,)), ((), ())),
                          preferred_element_type=jnp.float32)
    sim3 = sim.reshape(blk // 8, 8, q)
    m_blk = jnp.max(sim3, axis=0)
    eq = sim3 == m_blk[None]
    j_iota = lax.broadcasted_iota(jnp.int32, sim3.shape, 0)
    j_min = jnp.min(jnp.where(eq, j_iota, _BIG), axis=0)
    s_iota = lax.broadcasted_iota(jnp.int32, (8, q), 0)
    rows_blk = i * blk + j_min * 8 + s_iota

    m_old = m_ref[...]
    upd = m_blk > m_old
    m_ref[...] = jnp.where(upd, m_blk, m_old)
    ab_ref[...] = jnp.where(upd, rows_blk, ab_ref[...])

    @pl.when(i == nb - 1)
    def _fin():
        m = m_ref[...]
        ab = ab_ref[...]
        e_conf = jnp.max(m, axis=0, keepdims=True)
        cand = jnp.where(m == e_conf, ab, _BIG)
        e_idx = jnp.min(cand, axis=0, keepdims=True)

        rc = rc_ref[...]
        rn = jnp.sqrt(jnp.sum(rc * rc, axis=1, keepdims=True))
        rcn = rc / jnp.maximum(rn, _EPS)
        rsim = lax.dot_general(rcn, resn_ref[...], (((1,), (1,)), ((), ())),
                               preferred_element_type=jnp.float32)
        r_conf = jnp.max(rsim, axis=0, keepdims=True)
        r_rows = lax.broadcasted_iota(jnp.int32, rsim.shape, 0)
        r_idx = jnp.min(jnp.where(rsim == r_conf, r_rows, _BIG), axis=0,
                        keepdims=True)

        ei_ref[...] = e_idx
        ri_ref[...] = r_idx
        conf = (e_conf > _THRESH) & (r_conf > _THRESH)
        sc_ref[...] = jnp.where(conf, _COEFF, 0.0).astype(jnp.float32)


def _stage_a(residuals, entity_centroids, relation_centroids):
    q, d = residuals.shape
    ne = entity_centroids.shape[0]
    nr = relation_centroids.shape[0]
    nb = ne // _BLK
    return pl.pallas_call(
        _topk_body,
        grid=(nb,),
        in_specs=[
            pl.BlockSpec((q, d), lambda i: (0, 0)),
            pl.BlockSpec((_BLK, d), lambda i: (i, 0)),
            pl.BlockSpec((nr, d), lambda i: (0, 0)),
        ],
        out_specs=[
            pl.BlockSpec((1, q), lambda i: (0, 0)),
            pl.BlockSpec((1, q), lambda i: (0, 0)),
            pl.BlockSpec((1, q), lambda i: (0, 0)),
        ],
        out_shape=[
            jax.ShapeDtypeStruct((1, q), jnp.int32),
            jax.ShapeDtypeStruct((1, q), jnp.int32),
            jax.ShapeDtypeStruct((1, q), jnp.float32),
        ],
        scratch_shapes=[
            pltpu.VMEM((8, q), jnp.float32),
            pltpu.VMEM((8, q), jnp.int32),
            pltpu.VMEM((q, d), jnp.float32),
        ],
    )(residuals, entity_centroids, relation_centroids)


@functools.lru_cache(maxsize=None)
def _make_gather(ne, nr, d, q):
    info = plsc.get_sparse_core_info()
    nc, ns = info.num_cores, info.num_subcores
    nw = nc * ns
    bpw = q // nw
    mesh = plsc.VectorSubcoreMesh(core_axis_name="c", subcore_axis_name="s")

    def body(table_hbm, mask_hbm, ei_hbm, rows_out, mg_out,
             ei_v, rows_v, mg_v, sem1, sem2):
        wid = lax.axis_index("s") * nc + lax.axis_index("c")
        base = wid * bpw
        pltpu.sync_copy(ei_hbm.at[pl.ds(base, bpw)], ei_v)
        chunks = [ei_v[pl.ds(c * 16, 16)] for c in range(bpw // 16)]
        copies = []
        for k in range(bpw):
            e = chunks[k // 16][k % 16]
            copies.append(pltpu.async_copy(table_hbm.at[e], rows_v.at[k], sem1))
            copies.append(pltpu.async_copy(mask_hbm.at[e], mg_v.at[k], sem2))
        for cp in copies:
            cp.wait()
        pltpu.sync_copy(rows_v, rows_out.at[pl.ds(base, bpw)])
        pltpu.sync_copy(mg_v, mg_out.at[pl.ds(base, bpw)])

    return pl.kernel(
        body,
        out_type=(jax.ShapeDtypeStruct((q, nr, d), jnp.float32),
                  jax.ShapeDtypeStruct((q, nr), jnp.float32)),
        mesh=mesh,
        scratch_types=[
            pltpu.VMEM((bpw,), jnp.int32),
            pltpu.VMEM((bpw, nr, d), jnp.float32),
            pltpu.VMEM((bpw, nr), jnp.float32),
            pltpu.SemaphoreType.DMA,
            pltpu.SemaphoreType.DMA,
        ],
    )


def _finalize_body(rows_ref, mg_ref, ri_ref, sc_ref, out_ref):
    q, nr, d = rows_ref.shape
    r_idx = ri_ref[...].reshape(q, 1)
    rows = rows_ref[...]
    r_iota = lax.broadcasted_iota(jnp.int32, (q, nr, 1), 1)
    sel = (r_iota == r_idx[:, :, None]).astype(jnp.float32)
    emb = jnp.sum(rows * sel, axis=1)
    mg = mg_ref[...]
    r_iota2 = lax.broadcasted_iota(jnp.int32, (q, nr), 1)
    mask = jnp.sum(jnp.where(r_iota2 == r_idx, mg, 0.0), axis=1, keepdims=True)
    scale = mask * sc_ref[...].reshape(q, 1)
    out_ref[...] = emb * scale


def _finalize(rows, mg, ri, sc):
    q, nr, d = rows.shape
    return pl.pallas_call(
        _finalize_body,
        out_shape=jax.ShapeDtypeStruct((q, d), jnp.float32),
    )(rows, mg, ri, sc)


def kernel(residuals, entity_centroids, relation_centroids, lookup_table,
           lookup_mask):
    q, d = residuals.shape
    ne, nr = lookup_mask.shape
    ei, ri, sc = _stage_a(residuals, entity_centroids, relation_centroids)
    rows, mg = _make_gather(ne, nr, d, q)(
        lookup_table, lookup_mask, ei.reshape(q))
    return _finalize(rows, mg, ri, sc)


# speedup vs baseline: 1.1311x; 1.1311x over previous
import functools

import jax
import jax.numpy as jnp
from jax import lax
from jax.experimental import pallas as pl
from jax.experimental.pallas import tpu as pltpu
from jax.experimental.pallas import tpu_sc as plsc

_COEFF = 0.5
_THRESH = 0.3
_EPS = 1e-12
_BLK = 2048
_BIG = 2**30


def _topk_body(resT_ref, ecT_ref, rc_ref, gi_ref, li_ref, ri_ref,
               sc_ref, m_ref, ab_ref, resnT_ref):
    i = pl.program_id(0)
    nb = pl.num_programs(0)
    blk = ecT_ref.shape[1]
    q = resT_ref.shape[1]
    nr = rc_ref.shape[0]

    @pl.when(i == 0)
    def _init():
        res = resT_ref[...]
        n = jnp.sqrt(jnp.sum(res * res, axis=0, keepdims=True))
        resnT_ref[...] = res / jnp.maximum(n, _EPS)
        m_ref[...] = jnp.full(m_ref.shape, -jnp.inf, jnp.float32)
        ab_ref[...] = jnp.zeros(ab_ref.shape, jnp.int32)

    ec = ecT_ref[...]
    n = jnp.sqrt(jnp.sum(ec * ec, axis=0, keepdims=True))
    ecn = ec / jnp.maximum(n, _EPS)
    resnT = resnT_ref[...]
    sim = lax.dot_general(ecn, resnT, (((0,), (0,)), ((), ())),
                          preferred_element_type=jnp.float32)
    sim3 = sim.reshape(blk // 8, 8, q)
    m_blk = jnp.max(sim3, axis=0)
    eq = sim3 == m_blk[None]
    j_iota = lax.broadcasted_iota(jnp.int32, sim3.shape, 0)
    j_min = jnp.min(jnp.where(eq, j_iota, _BIG), axis=0)
    s_iota = lax.broadcasted_iota(jnp.int32, (8, q), 0)
    rows_blk = i * blk + j_min * 8 + s_iota

    m_old = m_ref[...]
    upd = m_blk > m_old
    m_ref[...] = jnp.where(upd, m_blk, m_old)
    ab_ref[...] = jnp.where(upd, rows_blk, ab_ref[...])

    @pl.when(i == nb - 1)
    def _fin():
        m = m_ref[...]
        ab = ab_ref[...]
        e_conf = jnp.max(m, axis=0, keepdims=True)
        cand = jnp.where(m == e_conf, ab, _BIG)
        e_idx = jnp.min(cand, axis=0, keepdims=True)

        rc = rc_ref[...]
        rn = jnp.sqrt(jnp.sum(rc * rc, axis=1, keepdims=True))
        rcn = rc / jnp.maximum(rn, _EPS)
        rsim = lax.dot_general(rcn, resnT_ref[...], (((1,), (0,)), ((), ())),
                               preferred_element_type=jnp.float32)
        r_conf = jnp.max(rsim, axis=0, keepdims=True)
        r_rows = lax.broadcasted_iota(jnp.int32, rsim.shape, 0)
        r_idx = jnp.min(jnp.where(rsim == r_conf, r_rows, _BIG), axis=0,
                        keepdims=True)

        gi_ref[...] = e_idx // 128
        li_ref[...] = lax.rem(e_idx, 128)
        ri_ref[...] = r_idx
        conf = (e_conf > _THRESH) & (r_conf > _THRESH)
        sc_ref[...] = jnp.where(conf, _COEFF, 0.0).astype(jnp.float32)


def _stage_a(resT, ecT, rc):
    d, q = resT.shape
    ne = ecT.shape[1]
    nr = rc.shape[0]
    nb = ne // _BLK
    return pl.pallas_call(
        _topk_body,
        grid=(nb,),
        in_specs=[
            pl.BlockSpec((d, q), lambda i: (0, 0)),
            pl.BlockSpec((d, _BLK), lambda i: (0, i)),
            pl.BlockSpec((nr, d), lambda i: (0, 0)),
        ],
        out_specs=[
            pl.BlockSpec((1, q), lambda i: (0, 0)),
            pl.BlockSpec((1, q), lambda i: (0, 0)),
            pl.BlockSpec((1, q), lambda i: (0, 0)),
            pl.BlockSpec((1, q), lambda i: (0, 0)),
        ],
        out_shape=[
            jax.ShapeDtypeStruct((1, q), jnp.int32),
            jax.ShapeDtypeStruct((1, q), jnp.int32),
            jax.ShapeDtypeStruct((1, q), jnp.int32),
            jax.ShapeDtypeStruct((1, q), jnp.float32),
        ],
        scratch_shapes=[
            pltpu.VMEM((8, q), jnp.float32),
            pltpu.VMEM((8, q), jnp.int32),
            pltpu.VMEM((d, q), jnp.float32),
        ],
    )(resT, ecT, rc)


@functools.lru_cache(maxsize=None)
def _make_gather(ne, nr, d, q):
    info = plsc.get_sparse_core_info()
    nc, ns = info.num_cores, info.num_subcores
    nw = nc * ns
    bpw = q // nw
    mesh = plsc.VectorSubcoreMesh(core_axis_name="c", subcore_axis_name="s")

    def body(tableT_hbm, maskT_hbm, gi_hbm, ri_hbm, rows_out, mg_out,
             gi_v, ri_v, sem1, sem2):
        wid = lax.axis_index("s") * nc + lax.axis_index("c")
        base = wid * bpw
        pltpu.sync_copy(gi_hbm.at[pl.ds(base, bpw)], gi_v)
        pltpu.sync_copy(ri_hbm.at[pl.ds(base, bpw)], ri_v)
        gch = [gi_v[pl.ds(c * 16, 16)] for c in range(bpw // 16)]
        rch = [ri_v[pl.ds(c * 16, 16)] for c in range(bpw // 16)]
        copies = []
        for k in range(bpw):
            g = gch[k // 16][k % 16]
            r = rch[k // 16][k % 16]
            off = pl.multiple_of(g * 128, 128)
            copies.append(pltpu.async_copy(
                tableT_hbm.at[r, :, pl.ds(off, 128)], rows_out.at[base + k],
                sem1))
            copies.append(pltpu.async_copy(
                maskT_hbm.at[r, pl.ds(off, 128)], mg_out.at[base + k], sem2))
        for cp in copies:
            cp.wait()

    return pl.kernel(
        body,
        out_type=(jax.ShapeDtypeStruct((q, d, 128), jnp.float32),
                  jax.ShapeDtypeStruct((q, 128), jnp.float32)),
        mesh=mesh,
        scratch_types=[
            pltpu.VMEM((bpw,), jnp.int32),
            pltpu.VMEM((bpw,), jnp.int32),
            pltpu.SemaphoreType.DMA,
            pltpu.SemaphoreType.DMA,
        ],
    )


def _finalize_body(rows_ref, mg_ref, li_ref, sc_ref, out_ref):
    q, d, _ = rows_ref.shape
    li = li_ref[...].reshape(q, 1)
    lane2 = lax.broadcasted_iota(jnp.int32, (q, 128), 1)
    sel2 = (lane2 == li).astype(jnp.float32)
    emb = jnp.sum(rows_ref[...] * sel2[:, None, :], axis=2)
    mask = jnp.sum(mg_ref[...] * sel2, axis=1, keepdims=True)
    scale = mask * sc_ref[...].reshape(q, 1)
    out_ref[...] = (emb * scale).T


def _finalize(rows, mg, li, sc):
    q, d, _ = rows.shape
    outT = pl.pallas_call(
        _finalize_body,
        out_shape=jax.ShapeDtypeStruct((d, q), jnp.float32),
    )(rows, mg, li, sc)
    return outT.T


def kernel(residuals, entity_centroids, relation_centroids, lookup_table,
           lookup_mask):
    q, d = residuals.shape
    ne, nr = lookup_mask.shape
    ecT = entity_centroids.T
    pad = (-ne) % _BLK
    if pad:
        ecT = jnp.concatenate(
            [ecT, jnp.broadcast_to(ecT[:, ne - 1:ne], (d, pad))], axis=1)
    gi, li, ri, sc = _stage_a(residuals.T, ecT, relation_centroids)
    tableT = jnp.transpose(lookup_table, (1, 2, 0))
    maskT = lookup_mask.T
    rows, mg = _make_gather(ne, nr, d, q)(
        tableT, maskT, gi.reshape(q), ri.reshape(q))
    return _finalize(rows, mg, li, sc)


# speedup vs baseline: 4.3208x; 3.8199x over previous
import functools

import jax
import jax.numpy as jnp
from jax import lax
from jax.experimental import pallas as pl
from jax.experimental.pallas import tpu as pltpu
from jax.experimental.pallas import tpu_sc as plsc

_COEFF = 0.5
_THRESH = 0.3
_EPS = 1e-12
_BLK = 2048
_BIG = 2**30


def _topk_body(resT_ref, ecT_ref, rc_ref, gi_ref, li_ref, ri_ref,
               sc_ref, m_ref, ab_ref, resnT_ref):
    i = pl.program_id(0)
    nb = pl.num_programs(0)
    blk = ecT_ref.shape[1]
    q = resT_ref.shape[1]
    nr = rc_ref.shape[0]

    @pl.when(i == 0)
    def _init():
        res = resT_ref[...]
        n = jnp.sqrt(jnp.sum(res * res, axis=0, keepdims=True))
        resnT_ref[...] = res / jnp.maximum(n, _EPS)
        m_ref[...] = jnp.full(m_ref.shape, -jnp.inf, jnp.float32)
        ab_ref[...] = jnp.zeros(ab_ref.shape, jnp.int32)

    ec = ecT_ref[...]
    n = jnp.sqrt(jnp.sum(ec * ec, axis=0, keepdims=True))
    ecn = ec / jnp.maximum(n, _EPS)
    resnT = resnT_ref[...]
    sim = lax.dot_general(ecn, resnT, (((0,), (0,)), ((), ())),
                          preferred_element_type=jnp.float32)
    sim3 = sim.reshape(blk // 8, 8, q)
    m_blk = jnp.max(sim3, axis=0)
    eq = sim3 == m_blk[None]
    j_iota = lax.broadcasted_iota(jnp.int32, sim3.shape, 0)
    j_min = jnp.min(jnp.where(eq, j_iota, _BIG), axis=0)
    s_iota = lax.broadcasted_iota(jnp.int32, (8, q), 0)
    rows_blk = i * blk + j_min * 8 + s_iota

    m_old = m_ref[...]
    upd = m_blk > m_old
    m_ref[...] = jnp.where(upd, m_blk, m_old)
    ab_ref[...] = jnp.where(upd, rows_blk, ab_ref[...])

    @pl.when(i == nb - 1)
    def _fin():
        m = m_ref[...]
        ab = ab_ref[...]
        e_conf = jnp.max(m, axis=0, keepdims=True)
        cand = jnp.where(m == e_conf, ab, _BIG)
        e_idx = jnp.min(cand, axis=0, keepdims=True)

        rc = rc_ref[...]
        rn = jnp.sqrt(jnp.sum(rc * rc, axis=1, keepdims=True))
        rcn = rc / jnp.maximum(rn, _EPS)
        rsim = lax.dot_general(rcn, resnT_ref[...], (((1,), (0,)), ((), ())),
                               preferred_element_type=jnp.float32)
        r_conf = jnp.max(rsim, axis=0, keepdims=True)
        r_rows = lax.broadcasted_iota(jnp.int32, rsim.shape, 0)
        r_idx = jnp.min(jnp.where(rsim == r_conf, r_rows, _BIG), axis=0,
                        keepdims=True)

        gi_ref[...] = e_idx // 128
        li_ref[...] = lax.rem(e_idx, 128)
        ri_ref[...] = r_idx
        conf = (e_conf > _THRESH) & (r_conf > _THRESH)
        sc_ref[...] = jnp.where(conf, _COEFF, 0.0).astype(jnp.float32)


def _stage_a(resT, ecT, rc):
    d, q = resT.shape
    ne = ecT.shape[1]
    nr = rc.shape[0]
    nb = ne // _BLK
    return pl.pallas_call(
        _topk_body,
        grid=(nb,),
        in_specs=[
            pl.BlockSpec((d, q), lambda i: (0, 0)),
            pl.BlockSpec((d, _BLK), lambda i: (0, i)),
            pl.BlockSpec((nr, d), lambda i: (0, 0)),
        ],
        out_specs=[
            pl.BlockSpec((1, q), lambda i: (0, 0)),
            pl.BlockSpec((1, q), lambda i: (0, 0)),
            pl.BlockSpec((1, q), lambda i: (0, 0)),
            pl.BlockSpec((1, q), lambda i: (0, 0)),
        ],
        out_shape=[
            jax.ShapeDtypeStruct((1, q), jnp.int32),
            jax.ShapeDtypeStruct((1, q), jnp.int32),
            jax.ShapeDtypeStruct((1, q), jnp.int32),
            jax.ShapeDtypeStruct((1, q), jnp.float32),
        ],
        scratch_shapes=[
            pltpu.VMEM((8, q), jnp.float32),
            pltpu.VMEM((8, q), jnp.int32),
            pltpu.VMEM((d, q), jnp.float32),
        ],
    )(resT, ecT, rc)


@functools.lru_cache(maxsize=None)
def _make_gather(ne, nr, d, q):
    info = plsc.get_sparse_core_info()
    nc, ns = info.num_cores, info.num_subcores
    nw = nc * ns
    bpw = q // nw
    mesh = plsc.VectorSubcoreMesh(core_axis_name="c", subcore_axis_name="s")

    nbuf = 24

    def body(tableT_hbm, maskT_hbm, gi_hbm, ri_hbm, rows_out, mg_out,
             gi_v, ri_v, tbuf, mbuf, sem1, sem2, sem3, sem4):
        wid = lax.axis_index("s") * nc + lax.axis_index("c")
        base = wid * bpw
        pltpu.sync_copy(gi_hbm.at[pl.ds(base, bpw)], gi_v)
        pltpu.sync_copy(ri_hbm.at[pl.ds(base, bpw)], ri_v)
        gch = [gi_v[pl.ds(c * 16, 16)] for c in range(bpw // 16)]
        rch = [ri_v[pl.ds(c * 16, 16)] for c in range(bpw // 16)]
        tcp = [None] * nbuf
        mcp = [None] * nbuf
        wt = [None] * nbuf
        wm = [None] * nbuf

        def fire(k):
            b = k % nbuf
            g = gch[k // 16][k % 16]
            r = rch[k // 16][k % 16]
            off = pl.multiple_of(g * 128, 128)
            tcp[b] = pltpu.async_copy(
                tableT_hbm.at[r, :, pl.ds(off, 128)], tbuf.at[b], sem1)
            mcp[b] = pltpu.async_copy(
                maskT_hbm.at[r, pl.ds(off, 128)], mbuf.at[b], sem2)

        for k in range(min(nbuf, bpw)):
            fire(k)
        for k in range(bpw):
            b = k % nbuf
            tcp[b].wait()
            mcp[b].wait()
            wt[b] = pltpu.async_copy(tbuf.at[b], rows_out.at[base + k], sem3)
            wm[b] = pltpu.async_copy(mbuf.at[b], mg_out.at[base + k], sem4)
            nk = k + nbuf
            if nk < bpw:
                wt[b].wait()
                wm[b].wait()
                wt[b] = None
                wm[b] = None
                fire(nk)
        for b in range(nbuf):
            if wt[b] is not None:
                wt[b].wait()
                wm[b].wait()

    return pl.kernel(
        body,
        out_type=(jax.ShapeDtypeStruct((q, d, 128), jnp.float32),
                  jax.ShapeDtypeStruct((q, 128), jnp.float32)),
        mesh=mesh,
        scratch_types=[
            pltpu.VMEM((bpw,), jnp.int32),
            pltpu.VMEM((bpw,), jnp.int32),
            pltpu.VMEM((nbuf, d, 128), jnp.float32),
            pltpu.VMEM((nbuf, 128), jnp.float32),
            pltpu.SemaphoreType.DMA,
            pltpu.SemaphoreType.DMA,
            pltpu.SemaphoreType.DMA,
            pltpu.SemaphoreType.DMA,
        ],
    )


def _finalize_body(rows_ref, mg_ref, li_ref, sc_ref, out_ref):
    q, d, _ = rows_ref.shape
    li = li_ref[...].reshape(q, 1)
    lane2 = lax.broadcasted_iota(jnp.int32, (q, 128), 1)
    sel2 = (lane2 == li).astype(jnp.float32)
    emb = jnp.sum(rows_ref[...] * sel2[:, None, :], axis=2)
    mask = jnp.sum(mg_ref[...] * sel2, axis=1, keepdims=True)
    scale = mask * sc_ref[...].reshape(q, 1)
    out_ref[...] = (emb * scale).T


def _finalize(rows, mg, li, sc):
    q, d, _ = rows.shape
    outT = pl.pallas_call(
        _finalize_body,
        out_shape=jax.ShapeDtypeStruct((d, q), jnp.float32),
    )(rows, mg, li, sc)
    return outT.T


def kernel(residuals, entity_centroids, relation_centroids, lookup_table,
           lookup_mask):
    q, d = residuals.shape
    ne, nr = lookup_mask.shape
    ecT = entity_centroids.T
    pad = (-ne) % _BLK
    if pad:
        ecT = jnp.concatenate(
            [ecT, jnp.broadcast_to(ecT[:, ne - 1:ne], (d, pad))], axis=1)
    gi, li, ri, sc = _stage_a(residuals.T, ecT, relation_centroids)
    tableT = jnp.transpose(lookup_table, (1, 2, 0))
    maskT = lookup_mask.T
    rows, mg = _make_gather(ne, nr, d, q)(
        tableT, maskT, gi.reshape(q), ri.reshape(q))
    return _finalize(rows, mg, li, sc)


# speedup vs baseline: 4.6009x; 1.0648x over previous
import functools

import jax
import jax.numpy as jnp
from jax import lax
from jax.experimental import pallas as pl
from jax.experimental.pallas import tpu as pltpu
from jax.experimental.pallas import tpu_sc as plsc

_COEFF = 0.5
_THRESH = 0.3
_EPS = 1e-12
_BLK = 2048
_BIG = 2**30


def _topk_body(resT_ref, ecT_ref, tail_ref, rc_ref, gi_ref, li_ref, ri_ref,
               sc_ref, m_ref, ab_ref, resnT_ref):
    i = pl.program_id(0)
    nb = pl.num_programs(0)
    blk = ecT_ref.shape[1]
    q = resT_ref.shape[1]
    nr = rc_ref.shape[0]

    @pl.when(i == 0)
    def _init():
        res = resT_ref[...]
        n = jnp.sqrt(jnp.sum(res * res, axis=0, keepdims=True))
        resnT_ref[...] = res / jnp.maximum(n, _EPS)
        m_ref[...] = jnp.full(m_ref.shape, -jnp.inf, jnp.float32)
        ab_ref[...] = jnp.zeros(ab_ref.shape, jnp.int32)

    ec = jnp.where(i == nb - 1, tail_ref[...], ecT_ref[...])
    n = jnp.sqrt(jnp.sum(ec * ec, axis=0, keepdims=True))
    ecn = ec / jnp.maximum(n, _EPS)
    resnT = resnT_ref[...]
    sim = lax.dot_general(ecn, resnT, (((0,), (0,)), ((), ())),
                          preferred_element_type=jnp.float32)
    sim3 = sim.reshape(blk // 8, 8, q)
    m_blk = jnp.max(sim3, axis=0)
    eq = sim3 == m_blk[None]
    j_iota = lax.broadcasted_iota(jnp.int32, sim3.shape, 0)
    j_min = jnp.min(jnp.where(eq, j_iota, _BIG), axis=0)
    s_iota = lax.broadcasted_iota(jnp.int32, (8, q), 0)
    rows_blk = i * blk + j_min * 8 + s_iota

    m_old = m_ref[...]
    upd = m_blk > m_old
    m_ref[...] = jnp.where(upd, m_blk, m_old)
    ab_ref[...] = jnp.where(upd, rows_blk, ab_ref[...])

    @pl.when(i == nb - 1)
    def _fin():
        m = m_ref[...]
        ab = ab_ref[...]
        e_conf = jnp.max(m, axis=0, keepdims=True)
        cand = jnp.where(m == e_conf, ab, _BIG)
        e_idx = jnp.min(cand, axis=0, keepdims=True)

        rc = rc_ref[...]
        rn = jnp.sqrt(jnp.sum(rc * rc, axis=1, keepdims=True))
        rcn = rc / jnp.maximum(rn, _EPS)
        rsim = lax.dot_general(rcn, resnT_ref[...], (((1,), (0,)), ((), ())),
                               preferred_element_type=jnp.float32)
        r_conf = jnp.max(rsim, axis=0, keepdims=True)
        r_rows = lax.broadcasted_iota(jnp.int32, rsim.shape, 0)
        r_idx = jnp.min(jnp.where(rsim == r_conf, r_rows, _BIG), axis=0,
                        keepdims=True)

        gi_ref[...] = e_idx // 128
        li_ref[...] = lax.rem(e_idx, 128)
        ri_ref[...] = r_idx
        conf = (e_conf > _THRESH) & (r_conf > _THRESH)
        sc_ref[...] = jnp.where(conf, _COEFF, 0.0).astype(jnp.float32)


def _stage_a(resT, ecT, tail, rc):
    d, q = resT.shape
    ne = ecT.shape[1]
    nr = rc.shape[0]
    nfull = ne // _BLK
    nb = nfull + 1
    return pl.pallas_call(
        _topk_body,
        grid=(nb,),
        in_specs=[
            pl.BlockSpec((d, q), lambda i: (0, 0)),
            pl.BlockSpec((d, _BLK), lambda i: (0, jnp.minimum(i, nfull - 1))),
            pl.BlockSpec((d, _BLK), lambda i: (0, 0)),
            pl.BlockSpec((nr, d), lambda i: (0, 0)),
        ],
        out_specs=[
            pl.BlockSpec((1, q), lambda i: (0, 0)),
            pl.BlockSpec((1, q), lambda i: (0, 0)),
            pl.BlockSpec((1, q), lambda i: (0, 0)),
            pl.BlockSpec((1, q), lambda i: (0, 0)),
        ],
        out_shape=[
            jax.ShapeDtypeStruct((1, q), jnp.int32),
            jax.ShapeDtypeStruct((1, q), jnp.int32),
            jax.ShapeDtypeStruct((1, q), jnp.int32),
            jax.ShapeDtypeStruct((1, q), jnp.float32),
        ],
        scratch_shapes=[
            pltpu.VMEM((8, q), jnp.float32),
            pltpu.VMEM((8, q), jnp.int32),
            pltpu.VMEM((d, q), jnp.float32),
        ],
    )(resT, ecT, tail, rc)


@functools.lru_cache(maxsize=None)
def _make_gather(ne, nr, d, q):
    info = plsc.get_sparse_core_info()
    nc, ns = info.num_cores, info.num_subcores
    nw = nc * ns
    bpw = q // nw
    mesh = plsc.VectorSubcoreMesh(core_axis_name="c", subcore_axis_name="s")

    nbuf = 24

    def body(tableT_hbm, maskT_hbm, gi_hbm, ri_hbm, rows_out, mg_out,
             gi_v, ri_v, tbuf, mbuf, sem1, sem2, sem3, sem4):
        wid = lax.axis_index("s") * nc + lax.axis_index("c")
        base = wid * bpw
        pltpu.sync_copy(gi_hbm.at[pl.ds(base, bpw)], gi_v)
        pltpu.sync_copy(ri_hbm.at[pl.ds(base, bpw)], ri_v)
        gch = [gi_v[pl.ds(c * 16, 16)] for c in range(bpw // 16)]
        rch = [ri_v[pl.ds(c * 16, 16)] for c in range(bpw // 16)]
        tcp = [None] * nbuf
        mcp = [None] * nbuf
        wt = [None] * nbuf
        wm = [None] * nbuf

        def fire(k):
            b = k % nbuf
            g = gch[k // 16][k % 16]
            r = rch[k // 16][k % 16]
            off = pl.multiple_of(g * 128, 128)
            tcp[b] = pltpu.async_copy(
                tableT_hbm.at[r, :, pl.ds(off, 128)], tbuf.at[b], sem1)
            mcp[b] = pltpu.async_copy(
                maskT_hbm.at[r, pl.ds(off, 128)], mbuf.at[b], sem2)

        for k in range(min(nbuf, bpw)):
            fire(k)
        for k in range(bpw):
            b = k % nbuf
            tcp[b].wait()
            mcp[b].wait()
            wt[b] = pltpu.async_copy(tbuf.at[b], rows_out.at[base + k], sem3)
            wm[b] = pltpu.async_copy(mbuf.at[b], mg_out.at[base + k], sem4)
            nk = k + nbuf
            if nk < bpw:
                wt[b].wait()
                wm[b].wait()
                wt[b] = None
                wm[b] = None
                fire(nk)
        for b in range(nbuf):
            if wt[b] is not None:
                wt[b].wait()
                wm[b].wait()

    return pl.kernel(
        body,
        out_type=(jax.ShapeDtypeStruct((q, d, 128), jnp.float32),
                  jax.ShapeDtypeStruct((q, 128), jnp.float32)),
        mesh=mesh,
        scratch_types=[
            pltpu.VMEM((bpw,), jnp.int32),
            pltpu.VMEM((bpw,), jnp.int32),
            pltpu.VMEM((nbuf, d, 128), jnp.float32),
            pltpu.VMEM((nbuf, 128), jnp.float32),
            pltpu.SemaphoreType.DMA,
            pltpu.SemaphoreType.DMA,
            pltpu.SemaphoreType.DMA,
            pltpu.SemaphoreType.DMA,
        ],
    )


def _finalize_body(rows_ref, mg_ref, li_ref, sc_ref, out_ref):
    qb, d, _ = rows_ref.shape
    li = li_ref[...].reshape(qb, 1)
    lane2 = lax.broadcasted_iota(jnp.int32, (qb, 128), 1)
    sel2 = (lane2 == li).astype(jnp.float32)
    emb = jnp.sum(rows_ref[...] * sel2[:, None, :], axis=2)
    mask = jnp.sum(mg_ref[...] * sel2, axis=1, keepdims=True)
    scale = mask * sc_ref[...].reshape(qb, 1)
    out_ref[...] = (emb * scale).T


def _finalize(rows, mg, li, sc):
    q, d, _ = rows.shape
    qb = 128
    outT = pl.pallas_call(
        _finalize_body,
        grid=(q // qb,),
        in_specs=[
            pl.BlockSpec((qb, d, 128), lambda i: (i, 0, 0)),
            pl.BlockSpec((qb, 128), lambda i: (i, 0)),
            pl.BlockSpec((1, qb), lambda i: (0, i)),
            pl.BlockSpec((1, qb), lambda i: (0, i)),
        ],
        out_specs=pl.BlockSpec((d, qb), lambda i: (0, i)),
        out_shape=jax.ShapeDtypeStruct((d, q), jnp.float32),
    )(rows, mg, li, sc)
    return outT.T


def kernel(residuals, entity_centroids, relation_centroids, lookup_table,
           lookup_mask):
    q, d = residuals.shape
    ne, nr = lookup_mask.shape
    ecT = entity_centroids.T
    nfull = ne // _BLK
    rem = ne - nfull * _BLK
    tail = jnp.concatenate(
        [ecT[:, nfull * _BLK:],
         jnp.broadcast_to(ecT[:, ne - 1:ne], (d, _BLK - rem))], axis=1)
    gi, li, ri, sc = _stage_a(residuals.T, ecT, tail, relation_centroids)
    tableT = jnp.transpose(lookup_table, (1, 2, 0))
    maskT = lookup_mask.T
    rows, mg = _make_gather(ne, nr, d, q)(
        tableT, maskT, gi.reshape(q), ri.reshape(q))
    return _finalize(rows, mg, li, sc)


# speedup vs baseline: 4.6842x; 1.0181x over previous
import functools

import jax
import jax.numpy as jnp
from jax import lax
from jax.experimental import pallas as pl
from jax.experimental.pallas import tpu as pltpu
from jax.experimental.pallas import tpu_sc as plsc

_COEFF = 0.5
_THRESH = 0.3
_EPS = 1e-12
_BLK = 4096
_BIG = 2**30


def _topk_body(resT_ref, ecT_ref, tail_ref, rc_ref, gi_ref, li_ref, ri_ref,
               sc_ref, m_ref, ab_ref, resnT_ref):
    i = pl.program_id(0)
    nb = pl.num_programs(0)
    blk = ecT_ref.shape[1]
    q = resT_ref.shape[1]
    nr = rc_ref.shape[0]

    @pl.when(i == 0)
    def _init():
        res = resT_ref[...]
        n = jnp.sqrt(jnp.sum(res * res, axis=0, keepdims=True))
        resnT_ref[...] = res / jnp.maximum(n, _EPS)
        m_ref[...] = jnp.full(m_ref.shape, -jnp.inf, jnp.float32)
        ab_ref[...] = jnp.zeros(ab_ref.shape, jnp.int32)

    ec = jnp.where(i == nb - 1, tail_ref[...], ecT_ref[...])
    n = jnp.sqrt(jnp.sum(ec * ec, axis=0, keepdims=True))
    ecn = ec / jnp.maximum(n, _EPS)
    resnT = resnT_ref[...]
    sim = lax.dot_general(ecn, resnT, (((0,), (0,)), ((), ())),
                          preferred_element_type=jnp.float32)
    sim3 = sim.reshape(blk // 8, 8, q)
    m_blk = jnp.max(sim3, axis=0)
    eq = sim3 == m_blk[None]
    j_iota = lax.broadcasted_iota(jnp.int32, sim3.shape, 0)
    j_min = jnp.min(jnp.where(eq, j_iota, _BIG), axis=0)
    s_iota = lax.broadcasted_iota(jnp.int32, (8, q), 0)
    rows_blk = i * blk + j_min * 8 + s_iota

    m_old = m_ref[...]
    upd = m_blk > m_old
    m_ref[...] = jnp.where(upd, m_blk, m_old)
    ab_ref[...] = jnp.where(upd, rows_blk, ab_ref[...])

    @pl.when(i == nb - 1)
    def _fin():
        m = m_ref[...]
        ab = ab_ref[...]
        e_conf = jnp.max(m, axis=0, keepdims=True)
        cand = jnp.where(m == e_conf, ab, _BIG)
        e_idx = jnp.min(cand, axis=0, keepdims=True)

        rc = rc_ref[...]
        rn = jnp.sqrt(jnp.sum(rc * rc, axis=1, keepdims=True))
        rcn = rc / jnp.maximum(rn, _EPS)
        rsim = lax.dot_general(rcn, resnT_ref[...], (((1,), (0,)), ((), ())),
                               preferred_element_type=jnp.float32)
        r_conf = jnp.max(rsim, axis=0, keepdims=True)
        r_rows = lax.broadcasted_iota(jnp.int32, rsim.shape, 0)
        r_idx = jnp.min(jnp.where(rsim == r_conf, r_rows, _BIG), axis=0,
                        keepdims=True)

        gi_ref[...] = e_idx // 128
        li_ref[...] = lax.rem(e_idx, 128)
        ri_ref[...] = r_idx
        conf = (e_conf > _THRESH) & (r_conf > _THRESH)
        sc_ref[...] = jnp.where(conf, _COEFF, 0.0).astype(jnp.float32)


def _stage_a(resT, ecT, tail, rc):
    d, q = resT.shape
    ne = ecT.shape[1]
    nr = rc.shape[0]
    nfull = ne // _BLK
    nb = nfull + 1
    return pl.pallas_call(
        _topk_body,
        grid=(nb,),
        in_specs=[
            pl.BlockSpec((d, q), lambda i: (0, 0)),
            pl.BlockSpec((d, _BLK), lambda i: (0, jnp.minimum(i, nfull - 1))),
            pl.BlockSpec((d, _BLK), lambda i: (0, 0)),
            pl.BlockSpec((nr, d), lambda i: (0, 0)),
        ],
        out_specs=[
            pl.BlockSpec((1, q), lambda i: (0, 0)),
            pl.BlockSpec((1, q), lambda i: (0, 0)),
            pl.BlockSpec((1, q), lambda i: (0, 0)),
            pl.BlockSpec((1, q), lambda i: (0, 0)),
        ],
        out_shape=[
            jax.ShapeDtypeStruct((1, q), jnp.int32),
            jax.ShapeDtypeStruct((1, q), jnp.int32),
            jax.ShapeDtypeStruct((1, q), jnp.int32),
            jax.ShapeDtypeStruct((1, q), jnp.float32),
        ],
        scratch_shapes=[
            pltpu.VMEM((8, q), jnp.float32),
            pltpu.VMEM((8, q), jnp.int32),
            pltpu.VMEM((d, q), jnp.float32),
        ],
    )(resT, ecT, tail, rc)


@functools.lru_cache(maxsize=None)
def _make_gather(ne, nr, d, q):
    info = plsc.get_sparse_core_info()
    nc, ns = info.num_cores, info.num_subcores
    nw = nc * ns
    bpw = q // nw
    mesh = plsc.VectorSubcoreMesh(core_axis_name="c", subcore_axis_name="s")

    nbuf = 24

    def body(tableT_hbm, maskT_hbm, gi_hbm, ri_hbm, rows_out, mg_out,
             gi_v, ri_v, tbuf, mbuf, sem1, sem2, sem3, sem4):
        wid = lax.axis_index("s") * nc + lax.axis_index("c")
        base = wid * bpw
        pltpu.sync_copy(gi_hbm.at[pl.ds(base, bpw)], gi_v)
        pltpu.sync_copy(ri_hbm.at[pl.ds(base, bpw)], ri_v)
        gch = [gi_v[pl.ds(c * 16, 16)] for c in range(bpw // 16)]
        rch = [ri_v[pl.ds(c * 16, 16)] for c in range(bpw // 16)]
        tcp = [None] * nbuf
        mcp = [None] * nbuf
        wt = [None] * nbuf
        wm = [None] * nbuf

        def fire(k):
            b = k % nbuf
            g = gch[k // 16][k % 16]
            r = rch[k // 16][k % 16]
            off = pl.multiple_of(g * 128, 128)
            tcp[b] = pltpu.async_copy(
                tableT_hbm.at[r, :, pl.ds(off, 128)], tbuf.at[b], sem1)
            mcp[b] = pltpu.async_copy(
                maskT_hbm.at[r, pl.ds(off, 128)], mbuf.at[b], sem2)

        for k in range(min(nbuf, bpw)):
            fire(k)
        for k in range(bpw):
            b = k % nbuf
            tcp[b].wait()
            mcp[b].wait()
            wt[b] = pltpu.async_copy(tbuf.at[b], rows_out.at[base + k], sem3)
            wm[b] = pltpu.async_copy(mbuf.at[b], mg_out.at[base + k], sem4)
            nk = k + nbuf
            if nk < bpw:
                wt[b].wait()
                wm[b].wait()
                wt[b] = None
                wm[b] = None
                fire(nk)
        for b in range(nbuf):
            if wt[b] is not None:
                wt[b].wait()
                wm[b].wait()

    return pl.kernel(
        body,
        out_type=(jax.ShapeDtypeStruct((q, d, 128), jnp.float32),
                  jax.ShapeDtypeStruct((q, 128), jnp.float32)),
        mesh=mesh,
        scratch_types=[
            pltpu.VMEM((bpw,), jnp.int32),
            pltpu.VMEM((bpw,), jnp.int32),
            pltpu.VMEM((nbuf, d, 128), jnp.float32),
            pltpu.VMEM((nbuf, 128), jnp.float32),
            pltpu.SemaphoreType.DMA,
            pltpu.SemaphoreType.DMA,
            pltpu.SemaphoreType.DMA,
            pltpu.SemaphoreType.DMA,
        ],
    )


def _finalize_body(rows_ref, mg_ref, li_ref, sc_ref, out_ref):
    qb, d, _ = rows_ref.shape
    li = li_ref[...].reshape(qb, 1)
    lane2 = lax.broadcasted_iota(jnp.int32, (qb, 128), 1)
    sel2 = (lane2 == li).astype(jnp.float32)
    emb = jnp.sum(rows_ref[...] * sel2[:, None, :], axis=2)
    mask = jnp.sum(mg_ref[...] * sel2, axis=1, keepdims=True)
    scale = mask * sc_ref[...].reshape(qb, 1)
    out_ref[...] = (emb * scale).T


def _finalize(rows, mg, li, sc):
    q, d, _ = rows.shape
    qb = 128
    outT = pl.pallas_call(
        _finalize_body,
        grid=(q // qb,),
        in_specs=[
            pl.BlockSpec((qb, d, 128), lambda i: (i, 0, 0)),
            pl.BlockSpec((qb, 128), lambda i: (i, 0)),
            pl.BlockSpec((1, qb), lambda i: (0, i)),
            pl.BlockSpec((1, qb), lambda i: (0, i)),
        ],
        out_specs=pl.BlockSpec((d, qb), lambda i: (0, i)),
        out_shape=jax.ShapeDtypeStruct((d, q), jnp.float32),
    )(rows, mg, li, sc)
    return outT.T


def kernel(residuals, entity_centroids, relation_centroids, lookup_table,
           lookup_mask):
    q, d = residuals.shape
    ne, nr = lookup_mask.shape
    ecT = entity_centroids.T
    nfull = ne // _BLK
    rem = ne - nfull * _BLK
    tail = jnp.concatenate(
        [ecT[:, nfull * _BLK:],
         jnp.broadcast_to(ecT[:, ne - 1:ne], (d, _BLK - rem))], axis=1)
    gi, li, ri, sc = _stage_a(residuals.T, ecT, tail, relation_centroids)
    tableT = jnp.transpose(lookup_table, (1, 2, 0))
    maskT = lookup_mask.T
    rows, mg = _make_gather(ne, nr, d, q)(
        tableT, maskT, gi.reshape(q), ri.reshape(q))
    return _finalize(rows, mg, li, sc)
